# baseline (speedup 1.0000x reference)
import functools

import jax
import jax.numpy as jnp
from jax import lax
from jax.experimental import pallas as pl
from jax.experimental.pallas import tpu as pltpu
from jax.experimental.pallas import tpu_sc as plsc

N = 10000
E = 320000
D = 128
NP = 10240
NC = 2
NS = 16
NW = NC * NS
K = 128
CH = 80
EPT = E // NW
RPT = NP // NS
RCH = RPT // K


NB = 2
NH = 2
CHH = CH // NH


def _zero_acc(s, zrow, acc_sh):
    for r in range(RCH):
        off = pl.multiple_of(s * RPT + r * K, K)
        pltpu.sync_copy(zrow, acc_sh.at[pl.ds(off, K)])
    plsc.subcore_barrier()


def _epilogue(c, s, acc_sh, rows_v, out):
    plsc.subcore_barrier()
    for r in range(RCH):
        off = pl.multiple_of(s * RPT + r * K, K)
        pltpu.sync_copy(acc_sh.at[pl.ds(off, K)], rows_v.at[0])
        pltpu.sync_copy(rows_v.at[0], out.at[c, pl.ds(off, K)])


def _scatter_body(table, src_idx, dst_idx, zrow, out, src_v, dst_v, rows_v,
                  acc_sh, *sems):
    gsem, ssem = sems[:NB], sems[NB:]
    c = lax.axis_index("c")
    s = lax.axis_index("s")
    wid = c * NS + s
    _zero_acc(s, zrow, acc_sh)

    def g_cp(j, b):
        return pltpu.make_async_copy(table.at[src_v.at[j]], rows_v.at[b],
                                     gsem[b])

    def s_cp(j, b):
        return pltpu.make_async_copy(rows_v.at[b], acc_sh.at[dst_v.at[j]],
                                     ssem[b])

    for half in range(NH):
        pltpu.sync_copy(src_idx.at[wid, pl.ds(half * CHH, CHH)], src_v)
        pltpu.sync_copy(dst_idx.at[wid, pl.ds(half * CHH, CHH)], dst_v)
        for b in range(NB):
            g_cp(b, b).start()

        def group(o, carry):
            for b in range(NB):
                j = o * NB + b
                g_cp(j, b).wait()
                pltpu.async_copy(rows_v.at[b], acc_sh.at[dst_v.at[j]],
                                 ssem[b], add=True)
            for b in range(NB):
                j = o * NB + b
                s_cp(j, b).wait()

                @pl.when(j + NB < CHH)
                def _():
                    g_cp(j + NB, b).start()
            return carry

        lax.fori_loop(0, CHH // NB, group, 0)
    _epilogue(c, s, acc_sh, rows_v, out)


def _scatter2_body(table_b, table_a, edges_idx, vert_idx, zrow, out,
                   e_v, v_v, rows_v, acc_sh, *sems):
    gsem, asem, ssem = sems[:NB], sems[NB:2 * NB], sems[2 * NB:]
    c = lax.axis_index("c")
    s = lax.axis_index("s")
    wid = c * NS + s
    _zero_acc(s, zrow, acc_sh)

    def g_cp(j, b):
        return pltpu.make_async_copy(table_b.at[e_v.at[j]], rows_v.at[b],
                                     gsem[b])

    def a_cp(j, b):
        return pltpu.make_async_copy(table_a.at[v_v.at[j]], rows_v.at[b],
                                     asem[b])

    def s_cp(j, b):
        return pltpu.make_async_copy(rows_v.at[b], acc_sh.at[v_v.at[j]],
                                     ssem[b])

    for half in range(NH):
        pltpu.sync_copy(edges_idx.at[wid, pl.ds(half * CHH, CHH)], e_v)
        pltpu.sync_copy(vert_idx.at[wid, pl.ds(half * CHH, CHH)], v_v)
        for b in range(NB):
            g_cp(b, b).start()

        def group(o, carry):
            for b in range(NB):
                j = o * NB + b
                g_cp(j, b).wait()
                pltpu.async_copy(table_a.at[v_v.at[j]], rows_v.at[b],
                                 asem[b], add=True)
            for b in range(NB):
                j = o * NB + b
                a_cp(j, b).wait()
                pltpu.async_copy(rows_v.at[b], acc_sh.at[v_v.at[j]],
                                 ssem[b], add=True)
            for b in range(NB):
                j = o * NB + b
                s_cp(j, b).wait()

                @pl.when(j + NB < CHH)
                def _():
                    g_cp(j + NB, b).start()
            return carry

        lax.fori_loop(0, CHH // NB, group, 0)
    _epilogue(c, s, acc_sh, rows_v, out)


@functools.lru_cache(maxsize=None)
def _make_scatter(two_tables):
    mesh = plsc.VectorSubcoreMesh(core_axis_name="c", subcore_axis_name="s",
                                  num_cores=NC, num_subcores=NS)
    return pl.kernel(
        _scatter2_body if two_tables else _scatter_body,
        out_type=jax.ShapeDtypeStruct((NC, NP, D), jnp.float32),
        mesh=mesh,
        scratch_types=[
            pltpu.VMEM((CHH, K), jnp.int32),
            pltpu.VMEM((CHH, K), jnp.int32),
            pltpu.VMEM((NB, K, D), jnp.float32),
            pltpu.VMEM_SHARED((NP, D), jnp.float32),
        ] + [pltpu.SemaphoreType.DMA] * ((3 if two_tables else 2) * NB),
    )


def _t0_body(x_ref, wl_ref, bl_ref, w1_ref, b1_ref, h_ref, g_ref):
    h = jnp.maximum(
        jnp.dot(x_ref[...], wl_ref[...], preferred_element_type=jnp.float32)
        + bl_ref[...], 0.0)
    h_ref[...] = h
    g_ref[...] = (jnp.dot(h, w1_ref[...], preferred_element_type=jnp.float32)
                  + b1_ref[...])


def _t1_body(p_ref, h_ref, w2a_ref, w2b_ref, b2_ref, a_ref, b_ref):
    xe = p_ref[0] + p_ref[1]
    a_ref[...] = (jnp.dot(h_ref[...], w2a_ref[...],
                          preferred_element_type=jnp.float32) + b2_ref[...])
    b_ref[...] = jnp.dot(xe, w2b_ref[...], preferred_element_type=jnp.float32)


def _mid_body(q_ref, h0_ref, w3_ref, b3_ref, w1_ref, b1_ref, h2_ref, g2_ref):
    xv = q_ref[0] + q_ref[1]
    u = 0.5 * xv + 0.5 * h0_ref[...]
    h2 = jnp.maximum(
        jnp.dot(u, w3_ref[...], preferred_element_type=jnp.float32)
        + b3_ref[...], 0.0)
    h2_ref[...] = h2
    g2_ref[...] = (jnp.dot(h2, w1_ref[...], preferred_element_type=jnp.float32)
                   + b1_ref[...])


def _final_body(q_ref, h0_ref, w3_ref, b3_ref, wc_ref, bc_ref, out_ref):
    xv = q_ref[0] + q_ref[1]
    u = 0.5 * xv + 0.5 * h0_ref[...]
    h3 = jnp.maximum(
        jnp.dot(u, w3_ref[...], preferred_element_type=jnp.float32)
        + b3_ref[...], 0.0)
    out_ref[...] = (jnp.dot(h3[:N], wc_ref[...],
                            preferred_element_type=jnp.float32) + bc_ref[...])


def _tc(body, out_shapes, *args):
    return pl.pallas_call(body, out_shape=out_shapes)(*args)


def kernel(x, edge_index, W_lin, b_lin, W1w, W1b, W2w, W2b, W3w, W3b, Wcw, Wcb):
    f32 = jnp.float32
    pad = jnp.full((NW, CH * K - EPT), N, jnp.int32)
    vertex = jnp.concatenate(
        [edge_index[0].reshape(NW, EPT), pad], axis=1).reshape(NW, CH, K)
    edges = jnp.concatenate(
        [edge_index[1].reshape(NW, EPT), pad], axis=1).reshape(NW, CH, K)
    xp = jnp.pad(x, ((0, NP - N), (0, 0)))
    zrow = jnp.zeros((K, D), f32)
    W2a, W2bb = W2w[:D], W2w[D:]
    bl = b_lin.reshape(1, D)
    b1 = W1b.reshape(1, D)
    b2 = W2b.reshape(1, D)
    b3 = W3b.reshape(1, D)
    bc = Wcb.reshape(1, -1)

    nd = jax.ShapeDtypeStruct((NP, D), f32)
    scat = _make_scatter(False)
    scat2 = _make_scatter(True)

    h0, g1 = _tc(_t0_body, (nd, nd), xp, W_lin, bl, W1w, b1)

    p1 = scat(g1, vertex, edges, zrow)
    a1, bt1 = _tc(_t1_body, (nd, nd), p1, h0, W2a, W2bb, b2)
    q1 = scat2(bt1, a1, edges, vertex, zrow)
    h2, g2 = _tc(_mid_body, (nd, nd), q1, h0, W3w, b3, W1w, b1)

    p2 = scat(g2, vertex, edges, zrow)
    a2, bt2 = _tc(_t1_body, (nd, nd), p2, h2, W2a, W2bb, b2)
    q2 = scat2(bt2, a2, edges, vertex, zrow)
    out = _tc(_final_body, jax.ShapeDtypeStruct((N, Wcw.shape[1]), f32),
              q2, h0, W3w, b3, Wcw, bc)
    return out


# speedup vs baseline: 1.2117x; 1.2117x over previous
import functools

import jax
import jax.numpy as jnp
from jax import lax
from jax.experimental import pallas as pl
from jax.experimental.pallas import tpu as pltpu
from jax.experimental.pallas import tpu_sc as plsc

N = 10000
E = 320000
D = 128
NP = 10240
NC = 2
NS = 16
NW = NC * NS
K = 128
CH = 80
EPT = E // NW
RPT = NP // NS
RCH = RPT // K
NH = 2
CHH = CH // NH
HR = NP // K


def _scatter_body(table, src_idx, dst_idx, zrow, out, src_v, dst_v, rows_v,
                  acc_sh, sem):
    c = lax.axis_index("c")
    s = lax.axis_index("s")
    wid = c * NS + s

    for r in range(RCH):
        off = pl.multiple_of(s * RPT + r * K, K)
        pltpu.sync_copy(zrow, acc_sh.at[pl.ds(off, K)])
    plsc.subcore_barrier()

    for half in range(NH):
        pltpu.sync_copy(src_idx.at[wid, pl.ds(half * CHH, CHH)], src_v)
        pltpu.sync_copy(dst_idx.at[wid, pl.ds(half * CHH, CHH)], dst_v)

        def chunk(j, carry):
            pltpu.async_copy(table.at[src_v.at[j]], rows_v, sem).wait()
            pltpu.sync_copy(rows_v, acc_sh.at[dst_v.at[j]], add=True)
            return carry

        lax.fori_loop(0, CHH, chunk, 0)

    plsc.subcore_barrier()
    for r in range(RCH):
        off = pl.multiple_of(s * RPT + r * K, K)
        pltpu.sync_copy(acc_sh.at[pl.ds(off, K)], rows_v)
        pltpu.sync_copy(rows_v, out.at[c, pl.ds(off, K)])


def _hist_body(vert_idx, out, idx_v, hacc):
    c = lax.axis_index("c")
    s = lax.axis_index("s")
    wid = c * NS + s
    zeros16 = jnp.zeros((16,), jnp.float32)
    ones16 = jnp.ones((16,), jnp.float32)

    def zrow(r, carry):
        for g in range(K // 16):
            hacc[r, pl.ds(g * 16, 16)] = zeros16
        return carry

    lax.fori_loop(0, HR, zrow, 0)

    for half in range(NH):
        pltpu.sync_copy(vert_idx.at[wid, pl.ds(half * CHH, CHH)], idx_v)

        def chunk(j, carry):
            for g in range(K // 16):
                iv = idx_v[j, pl.ds(g * 16, 16)]
                row = lax.shift_right_logical(iv, 7)
                col = lax.bitwise_and(iv, 127)
                plsc.addupdate_scatter(hacc, [row, col], ones16)
            return carry

        lax.fori_loop(0, CHH, chunk, 0)

    pltpu.sync_copy(hacc, out.at[wid])


@functools.lru_cache(maxsize=None)
def _mesh():
    return plsc.VectorSubcoreMesh(core_axis_name="c", subcore_axis_name="s",
                                  num_cores=NC, num_subcores=NS)


@functools.lru_cache(maxsize=None)
def _make_scatter():
    return pl.kernel(
        _scatter_body,
        out_type=jax.ShapeDtypeStruct((NC, NP, D), jnp.float32),
        mesh=_mesh(),
        scratch_types=[
            pltpu.VMEM((CHH, K), jnp.int32),
            pltpu.VMEM((CHH, K), jnp.int32),
            pltpu.VMEM((K, D), jnp.float32),
            pltpu.VMEM_SHARED((NP, D), jnp.float32),
            pltpu.SemaphoreType.DMA,
        ],
    )


@functools.lru_cache(maxsize=None)
def _make_hist():
    return pl.kernel(
        _hist_body,
        out_type=jax.ShapeDtypeStruct((NW, HR, K), jnp.float32),
        mesh=_mesh(),
        scratch_types=[
            pltpu.VMEM((CHH, K), jnp.int32),
            pltpu.VMEM((HR, K), jnp.float32),
        ],
        compiler_params=pltpu.CompilerParams(needs_layout_passes=False),
    )


def _t0_body(x_ref, wl_ref, bl_ref, w1_ref, b1_ref, w2a_ref, w2b_ref, b2_ref,
             h_ref, gb_ref, a_ref):
    h = jnp.maximum(
        jnp.dot(x_ref[...], wl_ref[...], preferred_element_type=jnp.float32)
        + bl_ref[...], 0.0)
    h_ref[...] = h
    g = jnp.dot(h, w1_ref[...], preferred_element_type=jnp.float32) + b1_ref[...]
    gb_ref[...] = jnp.dot(g, w2b_ref[...], preferred_element_type=jnp.float32)
    a_ref[...] = (jnp.dot(h, w2a_ref[...], preferred_element_type=jnp.float32)
                  + b2_ref[...])


def _hsum_body(p_ref, hist_ref, bsum_ref, cnt_ref):
    bsum_ref[...] = p_ref[0] + p_ref[1]
    cnt_ref[...] = jnp.sum(hist_ref[...], axis=0)


def _bsum_body(p_ref, bsum_ref):
    bsum_ref[...] = p_ref[0] + p_ref[1]


def _mid_body(q_ref, cnt_ref, a_ref, h0_ref, w3_ref, b3_ref, w1_ref, b1_ref,
              w2a_ref, w2b_ref, b2_ref, gb_ref, a2_ref):
    xv = q_ref[0] + q_ref[1] + cnt_ref[...] * a_ref[...]
    u = 0.5 * xv + 0.5 * h0_ref[...]
    h2 = jnp.maximum(
        jnp.dot(u, w3_ref[...], preferred_element_type=jnp.float32)
        + b3_ref[...], 0.0)
    g = jnp.dot(h2, w1_ref[...], preferred_element_type=jnp.float32) + b1_ref[...]
    gb_ref[...] = jnp.dot(g, w2b_ref[...], preferred_element_type=jnp.float32)
    a2_ref[...] = (jnp.dot(h2, w2a_ref[...], preferred_element_type=jnp.float32)
                   + b2_ref[...])


def _final_body(q_ref, cnt_ref, a_ref, h0_ref, w3_ref, b3_ref, wc_ref, bc_ref,
                out_ref):
    xv = q_ref[0] + q_ref[1] + cnt_ref[...] * a_ref[...]
    u = 0.5 * xv + 0.5 * h0_ref[...]
    h3 = jnp.maximum(
        jnp.dot(u, w3_ref[...], preferred_element_type=jnp.float32)
        + b3_ref[...], 0.0)
    out_ref[...] = (jnp.dot(h3[:N], wc_ref[...],
                            preferred_element_type=jnp.float32) + bc_ref[...])


def _tc(body, out_shapes, *args):
    return pl.pallas_call(body, out_shape=out_shapes)(*args)


def kernel(x, edge_index, W_lin, b_lin, W1w, W1b, W2w, W2b, W3w, W3b, Wcw, Wcb):
    f32 = jnp.float32
    pad = jnp.full((NW, CH * K - EPT), N, jnp.int32)
    vertex = jnp.concatenate(
        [edge_index[0].reshape(NW, EPT), pad], axis=1).reshape(NW, CH, K)
    edges = jnp.concatenate(
        [edge_index[1].reshape(NW, EPT), pad], axis=1).reshape(NW, CH, K)
    xp = jnp.pad(x, ((0, NP - N), (0, 0)))
    zrow = jnp.zeros((K, D), f32)
    W2a, W2bb = W2w[:D], W2w[D:]
    bl = b_lin.reshape(1, D)
    b1 = W1b.reshape(1, D)
    b2 = W2b.reshape(1, D)
    b3 = W3b.reshape(1, D)
    bc = Wcb.reshape(1, -1)

    nd = jax.ShapeDtypeStruct((NP, D), f32)
    scat = _make_scatter()

    h0, gb1, a1 = _tc(_t0_body, (nd, nd, nd),
                      xp, W_lin, bl, W1w, b1, W2a, W2bb, b2)
    hist = _make_hist()(vertex)

    p1 = scat(gb1, vertex, edges, zrow)
    bsum1, cntm = _tc(_hsum_body,
                      (nd, jax.ShapeDtypeStruct((HR, K), f32)), p1, hist)
    cnt_col = cntm.reshape(NP, 1)
    q1 = scat(bsum1, edges, vertex, zrow)
    gb2, a2 = _tc(_mid_body, (nd, nd),
                  q1, cnt_col, a1, h0, W3w, b3, W1w, b1, W2a, W2bb, b2)

    p2 = scat(gb2, vertex, edges, zrow)
    bsum2 = _tc(_bsum_body, nd, p2)
    q2 = scat(bsum2, edges, vertex, zrow)
    out = _tc(_final_body, jax.ShapeDtypeStruct((N, Wcw.shape[1]), f32),
              q2, cnt_col, a2, h0, W3w, b3, Wcw, bc)
    return out


# speedup vs baseline: 2.3870x; 1.9700x over previous
import functools

import jax
import jax.numpy as jnp
from jax import lax
from jax.experimental import pallas as pl
from jax.experimental.pallas import tpu as pltpu
from jax.experimental.pallas import tpu_sc as plsc

N = 10000
E = 320000
D = 128
DH = 64
NP = 10240
NC = 2
NS = 16
NW = NC * NS
K = 128
EPS = E // NS
CHT = 160
NQ = 4
CHQ = CHT // NQ
RPT = NP // NS
RCH = RPT // K
HR = NP // K


def _row_off(s, r):
    return pl.multiple_of(s * RPT + r * K, K)


def _conv_body(gb_half, vert_idx, edge_idx, zrow, out, ev_v, ee_v, rows_v,
               t_sh, a_sh, sem):
    c = lax.axis_index("c")
    s = lax.axis_index("s")

    for r in range(RCH):
        off = _row_off(s, r)
        pltpu.sync_copy(gb_half.at[c, pl.ds(off, K)], t_sh.at[pl.ds(off, K)])
        pltpu.sync_copy(zrow, a_sh.at[pl.ds(off, K)])
    plsc.subcore_barrier()

    def phase(src_sh, src_idx, dst_sh, dst_idx):
        for q in range(NQ):
            pltpu.sync_copy(src_idx.at[s, pl.ds(q * CHQ, CHQ)], ev_v)
            pltpu.sync_copy(dst_idx.at[s, pl.ds(q * CHQ, CHQ)], ee_v)

            def chunk(j, carry):
                pltpu.async_copy(src_sh.at[ev_v.at[j]], rows_v, sem).wait()
                pltpu.sync_copy(rows_v, dst_sh.at[ee_v.at[j]], add=True)
                return carry

            lax.fori_loop(0, CHQ, chunk, 0)

    phase(t_sh, vert_idx, a_sh, edge_idx)
    plsc.subcore_barrier()

    for r in range(RCH):
        pltpu.sync_copy(zrow, t_sh.at[pl.ds(_row_off(s, r), K)])
    plsc.subcore_barrier()

    phase(a_sh, edge_idx, t_sh, vert_idx)
    plsc.subcore_barrier()

    for r in range(RCH):
        off = _row_off(s, r)
        pltpu.sync_copy(t_sh.at[pl.ds(off, K)], out.at[c, pl.ds(off, K)])


def _hist_body(vert_idx, out, idx_v, hacc):
    c = lax.axis_index("c")
    s = lax.axis_index("s")
    zeros16 = jnp.zeros((16,), jnp.float32)
    ones16 = jnp.ones((16,), jnp.float32)

    def zrow(r, carry):
        for g in range(K // 16):
            hacc[r, pl.ds(g * 16, 16)] = zeros16
        return carry

    lax.fori_loop(0, HR, zrow, 0)

    for q in range(NQ // NC):
        qq = q * NC
        pltpu.sync_copy(
            vert_idx.at[s, pl.ds(pl.multiple_of((qq + c) * CHQ, CHQ), CHQ)],
            idx_v)

        def chunk(j, carry):
            for g in range(K // 16):
                iv = idx_v[j, pl.ds(g * 16, 16)]
                row = lax.shift_right_logical(iv, 7)
                col = lax.bitwise_and(iv, 127)
                plsc.addupdate_scatter(hacc, [row, col], ones16)
            return carry

        lax.fori_loop(0, CHQ, chunk, 0)

    wid = c * NS + s
    pltpu.sync_copy(hacc, out.at[wid])


@functools.lru_cache(maxsize=None)
def _mesh():
    return plsc.VectorSubcoreMesh(core_axis_name="c", subcore_axis_name="s",
                                  num_cores=NC, num_subcores=NS)


@functools.lru_cache(maxsize=None)
def _make_conv():
    return pl.kernel(
        _conv_body,
        out_type=jax.ShapeDtypeStruct((NC, NP, DH), jnp.float32),
        mesh=_mesh(),
        scratch_types=[
            pltpu.VMEM((CHQ, K), jnp.int32),
            pltpu.VMEM((CHQ, K), jnp.int32),
            pltpu.VMEM((K, DH), jnp.float32),
            pltpu.VMEM_SHARED((NP, DH), jnp.float32),
            pltpu.VMEM_SHARED((NP, DH), jnp.float32),
            pltpu.SemaphoreType.DMA,
        ],
        compiler_params=pltpu.CompilerParams(use_tc_tiling_on_sc=False),
    )


@functools.lru_cache(maxsize=None)
def _make_hist():
    return pl.kernel(
        _hist_body,
        out_type=jax.ShapeDtypeStruct((NW, HR, K), jnp.float32),
        mesh=_mesh(),
        scratch_types=[
            pltpu.VMEM((CHQ, K), jnp.int32),
            pltpu.VMEM((HR, K), jnp.float32),
        ],
        compiler_params=pltpu.CompilerParams(
            needs_layout_passes=False, use_tc_tiling_on_sc=False),
    )


def _t0_body(x_ref, wl_ref, bl_ref, w1_ref, b1_ref, w2a_ref, w2b_ref, b2_ref,
             h_ref, gb_ref, a_ref):
    h = jnp.maximum(
        jnp.dot(x_ref[...], wl_ref[...], preferred_element_type=jnp.float32)
        + bl_ref[...], 0.0)
    h_ref[...] = h
    g = jnp.dot(h, w1_ref[...], preferred_element_type=jnp.float32) + b1_ref[...]
    gb = jnp.dot(g, w2b_ref[...], preferred_element_type=jnp.float32)
    gb_ref[0] = gb[:, :DH]
    gb_ref[1] = gb[:, DH:]
    a_ref[...] = (jnp.dot(h, w2a_ref[...], preferred_element_type=jnp.float32)
                  + b2_ref[...])


def _hsum_body(hist_ref, cnt_ref):
    cnt_ref[...] = jnp.sum(hist_ref[...], axis=0)


def _mid_body(q_ref, cnt_ref, a_ref, h0_ref, w3_ref, b3_ref, w1_ref, b1_ref,
              w2a_ref, w2b_ref, b2_ref, gb_ref, a2_ref):
    z = jnp.concatenate([q_ref[0], q_ref[1]], axis=1)
    xv = z + cnt_ref[...] * a_ref[...]
    u = 0.5 * xv + 0.5 * h0_ref[...]
    h2 = jnp.maximum(
        jnp.dot(u, w3_ref[...], preferred_element_type=jnp.float32)
        + b3_ref[...], 0.0)
    g = jnp.dot(h2, w1_ref[...], preferred_element_type=jnp.float32) + b1_ref[...]
    gb = jnp.dot(g, w2b_ref[...], preferred_element_type=jnp.float32)
    gb_ref[0] = gb[:, :DH]
    gb_ref[1] = gb[:, DH:]
    a2_ref[...] = (jnp.dot(h2, w2a_ref[...], preferred_element_type=jnp.float32)
                   + b2_ref[...])


def _final_body(q_ref, cnt_ref, a_ref, h0_ref, w3_ref, b3_ref, wc_ref, bc_ref,
                out_ref):
    z = jnp.concatenate([q_ref[0], q_ref[1]], axis=1)
    xv = z + cnt_ref[...] * a_ref[...]
    u = 0.5 * xv + 0.5 * h0_ref[...]
    h3 = jnp.maximum(
        jnp.dot(u, w3_ref[...], preferred_element_type=jnp.float32)
        + b3_ref[...], 0.0)
    out_ref[...] = (jnp.dot(h3[:N], wc_ref[...],
                            preferred_element_type=jnp.float32) + bc_ref[...])


def _tc(body, out_shapes, *args):
    return pl.pallas_call(body, out_shape=out_shapes)(*args)


def kernel(x, edge_index, W_lin, b_lin, W1w, W1b, W2w, W2b, W3w, W3b, Wcw, Wcb):
    f32 = jnp.float32
    pad = jnp.full((NS, CHT * K - EPS), N, jnp.int32)
    vertex = jnp.concatenate(
        [edge_index[0].reshape(NS, EPS), pad], axis=1).reshape(NS, CHT, K)
    edges = jnp.concatenate(
        [edge_index[1].reshape(NS, EPS), pad], axis=1).reshape(NS, CHT, K)
    xp = jnp.pad(x, ((0, NP - N), (0, 0)))
    zrow = jnp.zeros((K, DH), f32)
    W2a, W2bb = W2w[:D], W2w[D:]
    bl = b_lin.reshape(1, D)
    b1 = W1b.reshape(1, D)
    b2 = W2b.reshape(1, D)
    b3 = W3b.reshape(1, D)
    bc = Wcb.reshape(1, -1)

    nd = jax.ShapeDtypeStruct((NP, D), f32)
    ndh = jax.ShapeDtypeStruct((NC, NP, DH), f32)
    conv = _make_conv()

    h0, gb1, a1 = _tc(_t0_body, (nd, ndh, nd),
                      xp, W_lin, bl, W1w, b1, W2a, W2bb, b2)
    hist = _make_hist()(vertex)
    cntm = _tc(_hsum_body, jax.ShapeDtypeStruct((HR, K), f32), hist)
    cnt_col = cntm.reshape(NP, 1)

    q1 = conv(gb1, vertex, edges, zrow)
    gb2, a2 = _tc(_mid_body, (ndh, nd),
                  q1, cnt_col, a1, h0, W3w, b3, W1w, b1, W2a, W2bb, b2)

    q2 = conv(gb2, vertex, edges, zrow)
    out = _tc(_final_body, jax.ShapeDtypeStruct((N, Wcw.shape[1]), f32),
              q2, cnt_col, a2, h0, W3w, b3, Wcw, bc)
    return out


# speedup vs baseline: 3.2309x; 1.3535x over previous
import functools

import jax
import jax.numpy as jnp
from jax import lax
from jax.experimental import pallas as pl
from jax.experimental.pallas import tpu as pltpu
from jax.experimental.pallas import tpu_sc as plsc

N = 10000
E = 320000
D = 128
DH = 64
NP = 10240
NC = 2
NS = 16
NW = NC * NS
K = 128
EPS = E // NS
CHT = 160
NQ = 4
CHQ = CHT // NQ
RPT = NP // NS
RCH = RPT // K
HR = NP // K


def _row_off(s, r):
    return pl.multiple_of(s * RPT + r * K, K)


def _conv_body(gb_half, vert_idx, edge_idx, zrow, out, iva, ivb, iea, ieb,
               rows_v, t_sh, a_sh, gsem0, gsem1, isem):
    c = lax.axis_index("c")
    s = lax.axis_index("s")
    gsem = (gsem0, gsem1)

    for r in range(RCH):
        off = _row_off(s, r)
        pltpu.sync_copy(gb_half.at[c, pl.ds(off, K)], t_sh.at[pl.ds(off, K)])
        pltpu.sync_copy(zrow, a_sh.at[pl.ds(off, K)])
    plsc.subcore_barrier()

    def phase(src_sh, src_idx, dst_sh, dst_idx):
        sbuf = (iva, ivb)
        dbuf = (iea, ieb)

        def i_cp(q, b):
            qo = pl.multiple_of(q * CHQ, CHQ)
            return (pltpu.make_async_copy(src_idx.at[s, pl.ds(qo, CHQ)],
                                          sbuf[b], isem),
                    pltpu.make_async_copy(dst_idx.at[s, pl.ds(qo, CHQ)],
                                          dbuf[b], isem))

        for cp in i_cp(0, 0):
            cp.start()
        for q in range(NQ):
            qb = q % 2
            for cp in i_cp(q, qb):
                cp.wait()
            if q + 1 < NQ:
                for cp in i_cp(q + 1, 1 - qb):
                    cp.start()
            src_v, dst_v = sbuf[qb], dbuf[qb]

            def g_cp(j, b):
                return pltpu.make_async_copy(src_sh.at[src_v.at[j]],
                                             rows_v.at[b], gsem[b])

            g_cp(0, 0).start()

            def pair(o, carry):
                for b in range(2):
                    j = o * 2 + b
                    g_cp(j, b).wait()

                    @pl.when(j + 1 < CHQ)
                    def _():
                        g_cp(j + 1, 1 - b).start()

                    pltpu.sync_copy(rows_v.at[b], dst_sh.at[dst_v.at[j]],
                                    add=True)
                return carry

            lax.fori_loop(0, CHQ // 2, pair, 0)

    phase(t_sh, vert_idx, a_sh, edge_idx)
    plsc.subcore_barrier()

    for r in range(RCH):
        pltpu.sync_copy(zrow, t_sh.at[pl.ds(_row_off(s, r), K)])
    plsc.subcore_barrier()

    phase(a_sh, edge_idx, t_sh, vert_idx)
    plsc.subcore_barrier()

    for r in range(RCH):
        off = _row_off(s, r)
        pltpu.sync_copy(t_sh.at[pl.ds(off, K)], out.at[c, pl.ds(off, K)])


def _hist_body(vert_idx, out, idx_v, hacc):
    c = lax.axis_index("c")
    s = lax.axis_index("s")
    zeros16 = jnp.zeros((16,), jnp.float32)
    ones16 = jnp.ones((16,), jnp.float32)

    def zrow(r, carry):
        for g in range(K // 16):
            hacc[r, pl.ds(g * 16, 16)] = zeros16
        return carry

    lax.fori_loop(0, HR, zrow, 0)

    for q in range(NQ // NC):
        qq = q * NC
        pltpu.sync_copy(
            vert_idx.at[s, pl.ds(pl.multiple_of((qq + c) * CHQ, CHQ), CHQ)],
            idx_v)

        def chunk(j, carry):
            for g in range(K // 16):
                iv = idx_v[j, pl.ds(g * 16, 16)]
                row = lax.shift_right_logical(iv, 7)
                col = lax.bitwise_and(iv, 127)
                plsc.addupdate_scatter(hacc, [row, col], ones16)
            return carry

        lax.fori_loop(0, CHQ, chunk, 0)

    wid = c * NS + s
    pltpu.sync_copy(hacc, out.at[wid])


@functools.lru_cache(maxsize=None)
def _mesh():
    return plsc.VectorSubcoreMesh(core_axis_name="c", subcore_axis_name="s",
                                  num_cores=NC, num_subcores=NS)


@functools.lru_cache(maxsize=None)
def _make_conv():
    return pl.kernel(
        _conv_body,
        out_type=jax.ShapeDtypeStruct((NC, NP, DH), jnp.float32),
        mesh=_mesh(),
        scratch_types=[
            pltpu.VMEM((CHQ, K), jnp.int32),
            pltpu.VMEM((CHQ, K), jnp.int32),
            pltpu.VMEM((CHQ, K), jnp.int32),
            pltpu.VMEM((CHQ, K), jnp.int32),
            pltpu.VMEM((2, K, DH), jnp.float32),
            pltpu.VMEM_SHARED((NP, DH), jnp.float32),
            pltpu.VMEM_SHARED((NP, DH), jnp.float32),
            pltpu.SemaphoreType.DMA,
            pltpu.SemaphoreType.DMA,
            pltpu.SemaphoreType.DMA,
        ],
        compiler_params=pltpu.CompilerParams(use_tc_tiling_on_sc=False),
    )


@functools.lru_cache(maxsize=None)
def _make_hist():
    return pl.kernel(
        _hist_body,
        out_type=jax.ShapeDtypeStruct((NW, HR, K), jnp.float32),
        mesh=_mesh(),
        scratch_types=[
            pltpu.VMEM((CHQ, K), jnp.int32),
            pltpu.VMEM((HR, K), jnp.float32),
        ],
        compiler_params=pltpu.CompilerParams(
            needs_layout_passes=False, use_tc_tiling_on_sc=False),
    )


def _t0_body(x_ref, wl_ref, bl_ref, w1_ref, b1_ref, w2a_ref, w2b_ref, b2_ref,
             h_ref, gb_ref, a_ref):
    h = jnp.maximum(
        jnp.dot(x_ref[...], wl_ref[...], preferred_element_type=jnp.float32)
        + bl_ref[...], 0.0)
    h_ref[...] = h
    g = jnp.dot(h, w1_ref[...], preferred_element_type=jnp.float32) + b1_ref[...]
    gb = jnp.dot(g, w2b_ref[...], preferred_element_type=jnp.float32)
    gb_ref[0] = gb[:, :DH]
    gb_ref[1] = gb[:, DH:]
    a_ref[...] = (jnp.dot(h, w2a_ref[...], preferred_element_type=jnp.float32)
                  + b2_ref[...])


def _hsum_body(hist_ref, cnt_ref):
    cnt_ref[...] = jnp.sum(hist_ref[...], axis=0)


def _mid_body(q_ref, cnt_ref, a_ref, h0_ref, w3_ref, b3_ref, w1_ref, b1_ref,
              w2a_ref, w2b_ref, b2_ref, gb_ref, a2_ref):
    z = jnp.concatenate([q_ref[0], q_ref[1]], axis=1)
    xv = z + cnt_ref[...] * a_ref[...]
    u = 0.5 * xv + 0.5 * h0_ref[...]
    h2 = jnp.maximum(
        jnp.dot(u, w3_ref[...], preferred_element_type=jnp.float32)
        + b3_ref[...], 0.0)
    g = jnp.dot(h2, w1_ref[...], preferred_element_type=jnp.float32) + b1_ref[...]
    gb = jnp.dot(g, w2b_ref[...], preferred_element_type=jnp.float32)
    gb_ref[0] = gb[:, :DH]
    gb_ref[1] = gb[:, DH:]
    a2_ref[...] = (jnp.dot(h2, w2a_ref[...], preferred_element_type=jnp.float32)
                   + b2_ref[...])


def _final_body(q_ref, cnt_ref, a_ref, h0_ref, w3_ref, b3_ref, wc_ref, bc_ref,
                out_ref):
    z = jnp.concatenate([q_ref[0], q_ref[1]], axis=1)
    xv = z + cnt_ref[...] * a_ref[...]
    u = 0.5 * xv + 0.5 * h0_ref[...]
    h3 = jnp.maximum(
        jnp.dot(u, w3_ref[...], preferred_element_type=jnp.float32)
        + b3_ref[...], 0.0)
    out_ref[...] = (jnp.dot(h3[:N], wc_ref[...],
                            preferred_element_type=jnp.float32) + bc_ref[...])


def _tc(body, out_shapes, *args):
    return pl.pallas_call(body, out_shape=out_shapes)(*args)


def kernel(x, edge_index, W_lin, b_lin, W1w, W1b, W2w, W2b, W3w, W3b, Wcw, Wcb):
    f32 = jnp.float32
    pad = jnp.full((NS, CHT * K - EPS), N, jnp.int32)
    vertex = jnp.concatenate(
        [edge_index[0].reshape(NS, EPS), pad], axis=1).reshape(NS, CHT, K)
    edges = jnp.concatenate(
        [edge_index[1].reshape(NS, EPS), pad], axis=1).reshape(NS, CHT, K)
    xp = jnp.pad(x, ((0, NP - N), (0, 0)))
    zrow = jnp.zeros((K, DH), f32)
    W2a, W2bb = W2w[:D], W2w[D:]
    bl = b_lin.reshape(1, D)
    b1 = W1b.reshape(1, D)
    b2 = W2b.reshape(1, D)
    b3 = W3b.reshape(1, D)
    bc = Wcb.reshape(1, -1)

    nd = jax.ShapeDtypeStruct((NP, D), f32)
    ndh = jax.ShapeDtypeStruct((NC, NP, DH), f32)
    conv = _make_conv()

    h0, gb1, a1 = _tc(_t0_body, (nd, ndh, nd),
                      xp, W_lin, bl, W1w, b1, W2a, W2bb, b2)
    hist = _make_hist()(vertex)
    cntm = _tc(_hsum_body, jax.ShapeDtypeStruct((HR, K), f32), hist)
    cnt_col = cntm.reshape(NP, 1)

    q1 = conv(gb1, vertex, edges, zrow)
    gb2, a2 = _tc(_mid_body, (ndh, nd),
                  q1, cnt_col, a1, h0, W3w, b3, W1w, b1, W2a, W2bb, b2)

    q2 = conv(gb2, vertex, edges, zrow)
    out = _tc(_final_body, jax.ShapeDtypeStruct((N, Wcw.shape[1]), f32),
              q2, cnt_col, a2, h0, W3w, b3, Wcw, bc)
    return out


# speedup vs baseline: 3.2341x; 1.0010x over previous
import functools

import jax
import jax.numpy as jnp
from jax import lax
from jax.experimental import pallas as pl
from jax.experimental.pallas import tpu as pltpu
from jax.experimental.pallas import tpu_sc as plsc

N = 10000
E = 320000
D = 128
DH = 64
NP = 10240
NC = 2
NS = 16
NW = NC * NS
K = 128
EPS = E // NS
CHT = 160
NQ = 4
CHQ = CHT // NQ
RPT = NP // NS
RCH = RPT // K
HR = NP // K


def _row_off(s, r):
    return pl.multiple_of(s * RPT + r * K, K)


NBUF = 2


def _conv_body(gb_half, vert_idx, edge_idx, zrow, out, iva, ivb, iea, ieb,
               rows_v, t_sh, a_sh, gsem0, gsem1, gsem2, gsem3, isem):
    c = lax.axis_index("c")
    s = lax.axis_index("s")
    gsem = (gsem0, gsem1, gsem2, gsem3)

    for r in range(RCH):
        off = _row_off(s, r)
        pltpu.sync_copy(gb_half.at[c, pl.ds(off, K)], t_sh.at[pl.ds(off, K)])
        pltpu.sync_copy(zrow, a_sh.at[pl.ds(off, K)])
    plsc.subcore_barrier()

    def phase(src_sh, src_idx, dst_sh, dst_idx):
        sbuf = (iva, ivb)
        dbuf = (iea, ieb)

        def i_cp(q, b):
            qo = pl.multiple_of(q * CHQ, CHQ)
            return (pltpu.make_async_copy(src_idx.at[s, pl.ds(qo, CHQ)],
                                          sbuf[b], isem),
                    pltpu.make_async_copy(dst_idx.at[s, pl.ds(qo, CHQ)],
                                          dbuf[b], isem))

        for cp in i_cp(0, 0):
            cp.start()
        for q in range(NQ):
            qb = q % 2
            for cp in i_cp(q, qb):
                cp.wait()
            if q + 1 < NQ:
                for cp in i_cp(q + 1, 1 - qb):
                    cp.start()
            src_v, dst_v = sbuf[qb], dbuf[qb]

            def g_cp(j, b):
                return pltpu.make_async_copy(src_sh.at[src_v.at[j]],
                                             rows_v.at[b], gsem[b])

            for b in range(NBUF - 1):
                g_cp(b, b).start()

            def grp(o, carry):
                for b in range(NBUF):
                    j = o * NBUF + b
                    g_cp(j, b).wait()
                    nxt = j + NBUF - 1

                    @pl.when(nxt < CHQ)
                    def _():
                        g_cp(nxt, (b + NBUF - 1) % NBUF).start()

                    pltpu.sync_copy(rows_v.at[b], dst_sh.at[dst_v.at[j]],
                                    add=True)
                return carry

            lax.fori_loop(0, CHQ // NBUF, grp, 0)

    phase(t_sh, vert_idx, a_sh, edge_idx)
    plsc.subcore_barrier()

    for r in range(RCH):
        pltpu.sync_copy(zrow, t_sh.at[pl.ds(_row_off(s, r), K)])
    plsc.subcore_barrier()

    phase(a_sh, edge_idx, t_sh, vert_idx)
    plsc.subcore_barrier()

    for r in range(RCH):
        off = _row_off(s, r)
        pltpu.sync_copy(t_sh.at[pl.ds(off, K)], out.at[c, pl.ds(off, K)])


def _hist_body(vert_idx, out, idx_v, hacc):
    c = lax.axis_index("c")
    s = lax.axis_index("s")
    zeros16 = jnp.zeros((16,), jnp.float32)
    ones16 = jnp.ones((16,), jnp.float32)

    def zrow(r, carry):
        for g in range(K // 16):
            hacc[r, pl.ds(g * 16, 16)] = zeros16
        return carry

    lax.fori_loop(0, HR, zrow, 0)

    for q in range(NQ // NC):
        qq = q * NC
        pltpu.sync_copy(
            vert_idx.at[s, pl.ds(pl.multiple_of((qq + c) * CHQ, CHQ), CHQ)],
            idx_v)

        def chunk(j, carry):
            for g in range(K // 16):
                iv = idx_v[j, pl.ds(g * 16, 16)]
                row = lax.shift_right_logical(iv, 7)
                col = lax.bitwise_and(iv, 127)
                plsc.addupdate_scatter(hacc, [row, col], ones16)
            return carry

        lax.fori_loop(0, CHQ, chunk, 0)

    wid = c * NS + s
    pltpu.sync_copy(hacc, out.at[wid])


@functools.lru_cache(maxsize=None)
def _mesh():
    return plsc.VectorSubcoreMesh(core_axis_name="c", subcore_axis_name="s",
                                  num_cores=NC, num_subcores=NS)


@functools.lru_cache(maxsize=None)
def _make_conv():
    return pl.kernel(
        _conv_body,
        out_type=jax.ShapeDtypeStruct((NC, NP, DH), jnp.float32),
        mesh=_mesh(),
        scratch_types=[
            pltpu.VMEM((CHQ, K), jnp.int32),
            pltpu.VMEM((CHQ, K), jnp.int32),
            pltpu.VMEM((CHQ, K), jnp.int32),
            pltpu.VMEM((CHQ, K), jnp.int32),
            pltpu.VMEM((NBUF, K, DH), jnp.float32),
            pltpu.VMEM_SHARED((NP, DH), jnp.float32),
            pltpu.VMEM_SHARED((NP, DH), jnp.float32),
        ] + [pltpu.SemaphoreType.DMA] * 5,
        compiler_params=pltpu.CompilerParams(use_tc_tiling_on_sc=False),
    )


@functools.lru_cache(maxsize=None)
def _make_hist():
    return pl.kernel(
        _hist_body,
        out_type=jax.ShapeDtypeStruct((NW, HR, K), jnp.float32),
        mesh=_mesh(),
        scratch_types=[
            pltpu.VMEM((CHQ, K), jnp.int32),
            pltpu.VMEM((HR, K), jnp.float32),
        ],
        compiler_params=pltpu.CompilerParams(
            needs_layout_passes=False, use_tc_tiling_on_sc=False),
    )


def _t0_body(x_ref, wl_ref, bl_ref, w1_ref, b1_ref, w2a_ref, w2b_ref, b2_ref,
             h_ref, gb_ref, a_ref):
    h = jnp.maximum(
        jnp.dot(x_ref[...], wl_ref[...], preferred_element_type=jnp.float32)
        + bl_ref[...], 0.0)
    h_ref[...] = h
    g = jnp.dot(h, w1_ref[...], preferred_element_type=jnp.float32) + b1_ref[...]
    gb = jnp.dot(g, w2b_ref[...], preferred_element_type=jnp.float32)
    gb_ref[0] = gb[:, :DH]
    gb_ref[1] = gb[:, DH:]
    a_ref[...] = (jnp.dot(h, w2a_ref[...], preferred_element_type=jnp.float32)
                  + b2_ref[...])


def _hsum_body(hist_ref, cnt_ref):
    cnt_ref[...] = jnp.sum(hist_ref[...], axis=0)


def _mid_body(q_ref, cnt_ref, a_ref, h0_ref, w3_ref, b3_ref, w1_ref, b1_ref,
              w2a_ref, w2b_ref, b2_ref, gb_ref, a2_ref):
    z = jnp.concatenate([q_ref[0], q_ref[1]], axis=1)
    xv = z + cnt_ref[...] * a_ref[...]
    u = 0.5 * xv + 0.5 * h0_ref[...]
    h2 = jnp.maximum(
        jnp.dot(u, w3_ref[...], preferred_element_type=jnp.float32)
        + b3_ref[...], 0.0)
    g = jnp.dot(h2, w1_ref[...], preferred_element_type=jnp.float32) + b1_ref[...]
    gb = jnp.dot(g, w2b_ref[...], preferred_element_type=jnp.float32)
    gb_ref[0] = gb[:, :DH]
    gb_ref[1] = gb[:, DH:]
    a2_ref[...] = (jnp.dot(h2, w2a_ref[...], preferred_element_type=jnp.float32)
                   + b2_ref[...])


def _final_body(q_ref, cnt_ref, a_ref, h0_ref, w3_ref, b3_ref, wc_ref, bc_ref,
                out_ref):
    z = jnp.concatenate([q_ref[0], q_ref[1]], axis=1)
    xv = z + cnt_ref[...] * a_ref[...]
    u = 0.5 * xv + 0.5 * h0_ref[...]
    h3 = jnp.maximum(
        jnp.dot(u, w3_ref[...], preferred_element_type=jnp.float32)
        + b3_ref[...], 0.0)
    out_ref[...] = (jnp.dot(h3[:N], wc_ref[...],
                            preferred_element_type=jnp.float32) + bc_ref[...])


def _tc(body, out_shapes, *args):
    return pl.pallas_call(body, out_shape=out_shapes)(*args)


def kernel(x, edge_index, W_lin, b_lin, W1w, W1b, W2w, W2b, W3w, W3b, Wcw, Wcb):
    f32 = jnp.float32
    pad = jnp.full((NS, CHT * K - EPS), N, jnp.int32)
    vertex = jnp.concatenate(
        [edge_index[0].reshape(NS, EPS), pad], axis=1).reshape(NS, CHT, K)
    edges = jnp.concatenate(
        [edge_index[1].reshape(NS, EPS), pad], axis=1).reshape(NS, CHT, K)
    xp = jnp.pad(x, ((0, NP - N), (0, 0)))
    zrow = jnp.zeros((K, DH), f32)
    W2a, W2bb = W2w[:D], W2w[D:]
    bl = b_lin.reshape(1, D)
    b1 = W1b.reshape(1, D)
    b2 = W2b.reshape(1, D)
    b3 = W3b.reshape(1, D)
    bc = Wcb.reshape(1, -1)

    nd = jax.ShapeDtypeStruct((NP, D), f32)
    ndh = jax.ShapeDtypeStruct((NC, NP, DH), f32)
    conv = _make_conv()

    h0, gb1, a1 = _tc(_t0_body, (nd, ndh, nd),
                      xp, W_lin, bl, W1w, b1, W2a, W2bb, b2)
    hist = _make_hist()(vertex)
    cntm = _tc(_hsum_body, jax.ShapeDtypeStruct((HR, K), f32), hist)
    cnt_col = cntm.reshape(NP, 1)

    q1 = conv(gb1, vertex, edges, zrow)
    gb2, a2 = _tc(_mid_body, (ndh, nd),
                  q1, cnt_col, a1, h0, W3w, b3, W1w, b1, W2a, W2bb, b2)

    q2 = conv(gb2, vertex, edges, zrow)
    out = _tc(_final_body, jax.ShapeDtypeStruct((N, Wcw.shape[1]), f32),
              q2, cnt_col, a2, h0, W3w, b3, Wcw, bc)
    return out


# speedup vs baseline: 3.4803x; 1.0761x over previous
import functools

import jax
import jax.numpy as jnp
from jax import lax
from jax.experimental import pallas as pl
from jax.experimental.pallas import tpu as pltpu
from jax.experimental.pallas import tpu_sc as plsc

N = 10000
E = 320000
D = 128
DH = 64
NP = 10240
NC = 2
NS = 16
NW = NC * NS
K = 128
EPS = E // NS
CHT = 160
NQ = 4
CHQ = CHT // NQ
RPT = NP // NS
RCH = RPT // K
HR = NP // K


def _row_off(s, r):
    return pl.multiple_of(s * RPT + r * K, K)


NBUF = 2


def _conv_body(gb_half, vert_idx, edge_idx, zrow, out, iva, ivb, iea, ieb,
               rows_v, t_sh, a_sh, gsem0, gsem1, gsem2, gsem3, isem):
    c = lax.axis_index("c")
    s = lax.axis_index("s")
    gsem = (gsem0, gsem1, gsem2, gsem3)
    col = pl.multiple_of(c * DH, 8)

    cps = []
    for r in range(RCH):
        off = _row_off(s, r)
        cps.append(pltpu.make_async_copy(
            gb_half.at[pl.ds(off, K), pl.ds(col, DH)],
            t_sh.at[pl.ds(off, K)], isem))
        cps.append(pltpu.make_async_copy(zrow, a_sh.at[pl.ds(off, K)], isem))
    for cp in cps:
        cp.start()
    for cp in cps:
        cp.wait()
    plsc.subcore_barrier()

    def phase(src_sh, src_idx, dst_sh, dst_idx):
        sbuf = (iva, ivb)
        dbuf = (iea, ieb)

        def i_cp(q, b):
            qo = pl.multiple_of(q * CHQ, CHQ)
            return (pltpu.make_async_copy(src_idx.at[s, pl.ds(qo, CHQ)],
                                          sbuf[b], isem),
                    pltpu.make_async_copy(dst_idx.at[s, pl.ds(qo, CHQ)],
                                          dbuf[b], isem))

        for cp in i_cp(0, 0):
            cp.start()
        for q in range(NQ):
            qb = q % 2
            for cp in i_cp(q, qb):
                cp.wait()
            if q + 1 < NQ:
                for cp in i_cp(q + 1, 1 - qb):
                    cp.start()
            src_v, dst_v = sbuf[qb], dbuf[qb]

            def g_cp(j, b):
                return pltpu.make_async_copy(src_sh.at[src_v.at[j]],
                                             rows_v.at[b], gsem[b])

            for b in range(NBUF - 1):
                g_cp(b, b).start()

            def grp(o, carry):
                for b in range(NBUF):
                    j = o * NBUF + b
                    g_cp(j, b).wait()
                    nxt = j + NBUF - 1

                    @pl.when(nxt < CHQ)
                    def _():
                        g_cp(nxt, (b + NBUF - 1) % NBUF).start()

                    pltpu.sync_copy(rows_v.at[b], dst_sh.at[dst_v.at[j]],
                                    add=True)
                return carry

            lax.fori_loop(0, CHQ // NBUF, grp, 0)

    phase(t_sh, vert_idx, a_sh, edge_idx)
    plsc.subcore_barrier()

    for r in range(RCH):
        pltpu.sync_copy(zrow, t_sh.at[pl.ds(_row_off(s, r), K)])
    plsc.subcore_barrier()

    phase(a_sh, edge_idx, t_sh, vert_idx)
    plsc.subcore_barrier()

    cps = []
    for r in range(RCH):
        off = _row_off(s, r)
        cps.append(pltpu.make_async_copy(
            t_sh.at[pl.ds(off, K)],
            out.at[pl.ds(off, K), pl.ds(col, DH)], isem))
    for cp in cps:
        cp.start()
    for cp in cps:
        cp.wait()


def _hist_body(vert_idx, out, idx_v, hacc):
    c = lax.axis_index("c")
    s = lax.axis_index("s")
    zeros16 = jnp.zeros((16,), jnp.float32)
    ones16 = jnp.ones((16,), jnp.float32)

    def zrow(r, carry):
        for g in range(K // 16):
            hacc[r, pl.ds(g * 16, 16)] = zeros16
        return carry

    lax.fori_loop(0, HR, zrow, 0)

    for q in range(NQ // NC):
        qq = q * NC
        pltpu.sync_copy(
            vert_idx.at[s, pl.ds(pl.multiple_of((qq + c) * CHQ, CHQ), CHQ)],
            idx_v)

        def chunk(j, carry):
            for g in range(K // 16):
                iv = idx_v[j, pl.ds(g * 16, 16)]
                row = lax.shift_right_logical(iv, 7)
                col = lax.bitwise_and(iv, 127)
                plsc.addupdate_scatter(hacc, [row, col], ones16)
            return carry

        lax.fori_loop(0, CHQ, chunk, 0)

    wid = c * NS + s
    pltpu.sync_copy(hacc, out.at[wid])


@functools.lru_cache(maxsize=None)
def _mesh():
    return plsc.VectorSubcoreMesh(core_axis_name="c", subcore_axis_name="s",
                                  num_cores=NC, num_subcores=NS)


@functools.lru_cache(maxsize=None)
def _make_conv():
    return pl.kernel(
        _conv_body,
        out_type=jax.ShapeDtypeStruct((NP, D), jnp.float32),
        mesh=_mesh(),
        scratch_types=[
            pltpu.VMEM((CHQ, K), jnp.int32),
            pltpu.VMEM((CHQ, K), jnp.int32),
            pltpu.VMEM((CHQ, K), jnp.int32),
            pltpu.VMEM((CHQ, K), jnp.int32),
            pltpu.VMEM((NBUF, K, DH), jnp.float32),
            pltpu.VMEM_SHARED((NP, DH), jnp.float32),
            pltpu.VMEM_SHARED((NP, DH), jnp.float32),
        ] + [pltpu.SemaphoreType.DMA] * 5,
        compiler_params=pltpu.CompilerParams(use_tc_tiling_on_sc=False),
    )


@functools.lru_cache(maxsize=None)
def _make_hist():
    return pl.kernel(
        _hist_body,
        out_type=jax.ShapeDtypeStruct((NW, HR, K), jnp.float32),
        mesh=_mesh(),
        scratch_types=[
            pltpu.VMEM((CHQ, K), jnp.int32),
            pltpu.VMEM((HR, K), jnp.float32),
        ],
        compiler_params=pltpu.CompilerParams(
            needs_layout_passes=False, use_tc_tiling_on_sc=False),
    )


def _t0_body(x_ref, wl_ref, bl_ref, w1_ref, b1_ref, w2a_ref, w2b_ref, b2_ref,
             h_ref, gb_ref, a_ref):
    h = jnp.maximum(
        jnp.dot(x_ref[...], wl_ref[...], preferred_element_type=jnp.float32)
        + bl_ref[...], 0.0)
    h_ref[...] = h
    g = jnp.dot(h, w1_ref[...], preferred_element_type=jnp.float32) + b1_ref[...]
    gb_ref[...] = jnp.dot(g, w2b_ref[...], preferred_element_type=jnp.float32)
    a_ref[...] = (jnp.dot(h, w2a_ref[...], preferred_element_type=jnp.float32)
                  + b2_ref[...])


def _hsum_body(hist_ref, cnt_ref):
    cnt_ref[...] = jnp.sum(hist_ref[...], axis=0)


def _mid_body(q_ref, cnt_ref, a_ref, h0_ref, w3_ref, b3_ref, w1_ref, b1_ref,
              w2a_ref, w2b_ref, b2_ref, gb_ref, a2_ref):
    xv = q_ref[...] + cnt_ref[...] * a_ref[...]
    u = 0.5 * xv + 0.5 * h0_ref[...]
    h2 = jnp.maximum(
        jnp.dot(u, w3_ref[...], preferred_element_type=jnp.float32)
        + b3_ref[...], 0.0)
    g = jnp.dot(h2, w1_ref[...], preferred_element_type=jnp.float32) + b1_ref[...]
    gb_ref[...] = jnp.dot(g, w2b_ref[...], preferred_element_type=jnp.float32)
    a2_ref[...] = (jnp.dot(h2, w2a_ref[...], preferred_element_type=jnp.float32)
                   + b2_ref[...])


def _final_body(q_ref, cnt_ref, a_ref, h0_ref, w3_ref, b3_ref, wc_ref, bc_ref,
                out_ref):
    xv = q_ref[...] + cnt_ref[...] * a_ref[...]
    u = 0.5 * xv + 0.5 * h0_ref[...]
    h3 = jnp.maximum(
        jnp.dot(u, w3_ref[...], preferred_element_type=jnp.float32)
        + b3_ref[...], 0.0)
    out_ref[...] = (jnp.dot(h3[:N], wc_ref[...],
                            preferred_element_type=jnp.float32) + bc_ref[...])


def _tc(body, out_shapes, *args):
    return pl.pallas_call(body, out_shape=out_shapes)(*args)


def kernel(x, edge_index, W_lin, b_lin, W1w, W1b, W2w, W2b, W3w, W3b, Wcw, Wcb):
    f32 = jnp.float32
    pad = jnp.full((NS, CHT * K - EPS), N, jnp.int32)
    vertex = jnp.concatenate(
        [edge_index[0].reshape(NS, EPS), pad], axis=1).reshape(NS, CHT, K)
    edges = jnp.concatenate(
        [edge_index[1].reshape(NS, EPS), pad], axis=1).reshape(NS, CHT, K)
    xp = jnp.pad(x, ((0, NP - N), (0, 0)))
    zrow = jnp.zeros((K, DH), f32)
    W2a, W2bb = W2w[:D], W2w[D:]
    bl = b_lin.reshape(1, D)
    b1 = W1b.reshape(1, D)
    b2 = W2b.reshape(1, D)
    b3 = W3b.reshape(1, D)
    bc = Wcb.reshape(1, -1)

    nd = jax.ShapeDtypeStruct((NP, D), f32)
    conv = _make_conv()

    h0, gb1, a1 = _tc(_t0_body, (nd, nd, nd),
                      xp, W_lin, bl, W1w, b1, W2a, W2bb, b2)
    hist = _make_hist()(vertex)
    cntm = _tc(_hsum_body, jax.ShapeDtypeStruct((HR, K), f32), hist)
    cnt_col = cntm.reshape(NP, 1)

    q1 = conv(gb1, vertex, edges, zrow)
    gb2, a2 = _tc(_mid_body, (nd, nd),
                  q1, cnt_col, a1, h0, W3w, b3, W1w, b1, W2a, W2bb, b2)

    q2 = conv(gb2, vertex, edges, zrow)
    out = _tc(_final_body, jax.ShapeDtypeStruct((N, Wcw.shape[1]), f32),
              q2, cnt_col, a2, h0, W3w, b3, Wcw, bc)
    return out


# speedup vs baseline: 3.6367x; 1.0449x over previous
import functools

import jax
import jax.numpy as jnp
from jax import lax
from jax.experimental import pallas as pl
from jax.experimental.pallas import tpu as pltpu
from jax.experimental.pallas import tpu_sc as plsc

N = 10000
E = 320000
D = 128
DH = 64
NP = 10240
NC = 2
NS = 16
NW = NC * NS
K = 128
EPS = E // NS
CHT = 160
NQ = 4
CHQ = CHT // NQ
RPT = NP // NS
RCH = RPT // K
HR = NP // K


def _row_off(s, r):
    return pl.multiple_of(s * RPT + r * K, K)


NBUF = 2


def _conv_body(gb_half, vert_idx, edge_idx, zrow, out, iva, ivb, iea, ieb,
               rows_v, t_sh, a_sh, gsem0, gsem1, gsem2, gsem3, isem):
    c = lax.axis_index("c")
    s = lax.axis_index("s")
    gsem = (gsem0, gsem1)
    ssem = (gsem2, gsem3)
    col = pl.multiple_of(c * DH, 8)

    cps = []
    for r in range(RCH):
        off = _row_off(s, r)
        cps.append(pltpu.make_async_copy(
            gb_half.at[pl.ds(off, K), pl.ds(col, DH)],
            t_sh.at[pl.ds(off, K)], isem))
        cps.append(pltpu.make_async_copy(zrow, a_sh.at[pl.ds(off, K)], isem))
    for cp in cps:
        cp.start()
    for cp in cps:
        cp.wait()
    plsc.subcore_barrier()

    def phase(src_sh, src_idx, dst_sh, dst_idx):
        sbuf = (iva, ivb)
        dbuf = (iea, ieb)

        def i_cp(q, b):
            qo = pl.multiple_of(q * CHQ, CHQ)
            return (pltpu.make_async_copy(src_idx.at[s, pl.ds(qo, CHQ)],
                                          sbuf[b], isem),
                    pltpu.make_async_copy(dst_idx.at[s, pl.ds(qo, CHQ)],
                                          dbuf[b], isem))

        for cp in i_cp(0, 0):
            cp.start()
        for q in range(NQ):
            qb = q % 2
            for cp in i_cp(q, qb):
                cp.wait()
            if q + 1 < NQ:
                for cp in i_cp(q + 1, 1 - qb):
                    cp.start()
            src_v, dst_v = sbuf[qb], dbuf[qb]

            def g_cp(j, b):
                return pltpu.make_async_copy(src_sh.at[src_v.at[j]],
                                             rows_v.at[b], gsem[b])

            def s_cp(j, b):
                return pltpu.make_async_copy(rows_v.at[b],
                                             dst_sh.at[dst_v.at[j]], ssem[b])

            g_cp(0, 0).start()

            def pair(o, carry):
                for b in range(2):
                    j = o * 2 + b
                    g_cp(j, b).wait()
                    pltpu.async_copy(rows_v.at[b], dst_sh.at[dst_v.at[j]],
                                     ssem[b], add=True)

                    @pl.when(j + 1 < CHQ)
                    def _():
                        @pl.when(j >= 1)
                        def _():
                            s_cp(j - 1, 1 - b).wait()

                        g_cp(j + 1, 1 - b).start()
                return carry

            lax.fori_loop(0, CHQ // 2, pair, 0)
            s_cp(CHQ - 2, 0).wait()
            s_cp(CHQ - 1, 1).wait()

    phase(t_sh, vert_idx, a_sh, edge_idx)
    plsc.subcore_barrier()

    for r in range(RCH):
        pltpu.sync_copy(zrow, t_sh.at[pl.ds(_row_off(s, r), K)])
    plsc.subcore_barrier()

    phase(a_sh, edge_idx, t_sh, vert_idx)
    plsc.subcore_barrier()

    cps = []
    for r in range(RCH):
        off = _row_off(s, r)
        cps.append(pltpu.make_async_copy(
            t_sh.at[pl.ds(off, K)],
            out.at[pl.ds(off, K), pl.ds(col, DH)], isem))
    for cp in cps:
        cp.start()
    for cp in cps:
        cp.wait()


def _hist_body(vert_idx, out, idx_v, hacc):
    c = lax.axis_index("c")
    s = lax.axis_index("s")
    zeros16 = jnp.zeros((16,), jnp.float32)
    ones16 = jnp.ones((16,), jnp.float32)

    def zrow(r, carry):
        for g in range(K // 16):
            hacc[r, pl.ds(g * 16, 16)] = zeros16
        return carry

    lax.fori_loop(0, HR, zrow, 0)

    for q in range(NQ // NC):
        qq = q * NC
        pltpu.sync_copy(
            vert_idx.at[s, pl.ds(pl.multiple_of((qq + c) * CHQ, CHQ), CHQ)],
            idx_v)

        def chunk(j, carry):
            for g in range(K // 16):
                iv = idx_v[j, pl.ds(g * 16, 16)]
                row = lax.shift_right_logical(iv, 7)
                col = lax.bitwise_and(iv, 127)
                plsc.addupdate_scatter(hacc, [row, col], ones16)
            return carry

        lax.fori_loop(0, CHQ, chunk, 0)

    wid = c * NS + s
    pltpu.sync_copy(hacc, out.at[wid])


@functools.lru_cache(maxsize=None)
def _mesh():
    return plsc.VectorSubcoreMesh(core_axis_name="c", subcore_axis_name="s",
                                  num_cores=NC, num_subcores=NS)


@functools.lru_cache(maxsize=None)
def _make_conv():
    return pl.kernel(
        _conv_body,
        out_type=jax.ShapeDtypeStruct((NP, D), jnp.float32),
        mesh=_mesh(),
        scratch_types=[
            pltpu.VMEM((CHQ, K), jnp.int32),
            pltpu.VMEM((CHQ, K), jnp.int32),
            pltpu.VMEM((CHQ, K), jnp.int32),
            pltpu.VMEM((CHQ, K), jnp.int32),
            pltpu.VMEM((NBUF, K, DH), jnp.float32),
            pltpu.VMEM_SHARED((NP, DH), jnp.float32),
            pltpu.VMEM_SHARED((NP, DH), jnp.float32),
        ] + [pltpu.SemaphoreType.DMA] * 5,
        compiler_params=pltpu.CompilerParams(use_tc_tiling_on_sc=False),
    )


@functools.lru_cache(maxsize=None)
def _make_hist():
    return pl.kernel(
        _hist_body,
        out_type=jax.ShapeDtypeStruct((NW, HR, K), jnp.float32),
        mesh=_mesh(),
        scratch_types=[
            pltpu.VMEM((CHQ, K), jnp.int32),
            pltpu.VMEM((HR, K), jnp.float32),
        ],
        compiler_params=pltpu.CompilerParams(
            needs_layout_passes=False, use_tc_tiling_on_sc=False),
    )


def _t0_body(x_ref, wl_ref, bl_ref, w1_ref, b1_ref, w2a_ref, w2b_ref, b2_ref,
             h_ref, gb_ref, a_ref):
    h = jnp.maximum(
        jnp.dot(x_ref[...], wl_ref[...], preferred_element_type=jnp.float32)
        + bl_ref[...], 0.0)
    h_ref[...] = h
    g = jnp.dot(h, w1_ref[...], preferred_element_type=jnp.float32) + b1_ref[...]
    gb_ref[...] = jnp.dot(g, w2b_ref[...], preferred_element_type=jnp.float32)
    a_ref[...] = (jnp.dot(h, w2a_ref[...], preferred_element_type=jnp.float32)
                  + b2_ref[...])


def _hsum_body(hist_ref, cnt_ref):
    cnt_ref[...] = jnp.sum(hist_ref[...], axis=0)


def _mid_body(q_ref, cnt_ref, a_ref, h0_ref, w3_ref, b3_ref, w1_ref, b1_ref,
              w2a_ref, w2b_ref, b2_ref, gb_ref, a2_ref):
    xv = q_ref[...] + cnt_ref[...] * a_ref[...]
    u = 0.5 * xv + 0.5 * h0_ref[...]
    h2 = jnp.maximum(
        jnp.dot(u, w3_ref[...], preferred_element_type=jnp.float32)
        + b3_ref[...], 0.0)
    g = jnp.dot(h2, w1_ref[...], preferred_element_type=jnp.float32) + b1_ref[...]
    gb_ref[...] = jnp.dot(g, w2b_ref[...], preferred_element_type=jnp.float32)
    a2_ref[...] = (jnp.dot(h2, w2a_ref[...], preferred_element_type=jnp.float32)
                   + b2_ref[...])


def _final_body(q_ref, cnt_ref, a_ref, h0_ref, w3_ref, b3_ref, wc_ref, bc_ref,
                out_ref):
    xv = q_ref[...] + cnt_ref[...] * a_ref[...]
    u = 0.5 * xv + 0.5 * h0_ref[...]
    h3 = jnp.maximum(
        jnp.dot(u, w3_ref[...], preferred_element_type=jnp.float32)
        + b3_ref[...], 0.0)
    out_ref[...] = (jnp.dot(h3[:N], wc_ref[...],
                            preferred_element_type=jnp.float32) + bc_ref[...])


def _tc(body, out_shapes, *args):
    return pl.pallas_call(body, out_shape=out_shapes)(*args)


def kernel(x, edge_index, W_lin, b_lin, W1w, W1b, W2w, W2b, W3w, W3b, Wcw, Wcb):
    f32 = jnp.float32
    pad = jnp.full((NS, CHT * K - EPS), N, jnp.int32)
    vertex = jnp.concatenate(
        [edge_index[0].reshape(NS, EPS), pad], axis=1).reshape(NS, CHT, K)
    edges = jnp.concatenate(
        [edge_index[1].reshape(NS, EPS), pad], axis=1).reshape(NS, CHT, K)
    xp = jnp.pad(x, ((0, NP - N), (0, 0)))
    zrow = jnp.zeros((K, DH), f32)
    W2a, W2bb = W2w[:D], W2w[D:]
    bl = b_lin.reshape(1, D)
    b1 = W1b.reshape(1, D)
    b2 = W2b.reshape(1, D)
    b3 = W3b.reshape(1, D)
    bc = Wcb.reshape(1, -1)

    nd = jax.ShapeDtypeStruct((NP, D), f32)
    conv = _make_conv()

    h0, gb1, a1 = _tc(_t0_body, (nd, nd, nd),
                      xp, W_lin, bl, W1w, b1, W2a, W2bb, b2)
    hist = _make_hist()(vertex)
    cntm = _tc(_hsum_body, jax.ShapeDtypeStruct((HR, K), f32), hist)
    cnt_col = cntm.reshape(NP, 1)

    q1 = conv(gb1, vertex, edges, zrow)
    gb2, a2 = _tc(_mid_body, (nd, nd),
                  q1, cnt_col, a1, h0, W3w, b3, W1w, b1, W2a, W2bb, b2)

    q2 = conv(gb2, vertex, edges, zrow)
    out = _tc(_final_body, jax.ShapeDtypeStruct((N, Wcw.shape[1]), f32),
              q2, cnt_col, a2, h0, W3w, b3, Wcw, bc)
    return out


# speedup vs baseline: 3.6415x; 1.0013x over previous
import functools

import jax
import jax.numpy as jnp
from jax import lax
from jax.experimental import pallas as pl
from jax.experimental.pallas import tpu as pltpu
from jax.experimental.pallas import tpu_sc as plsc

N = 10000
E = 320000
D = 128
DH = 64
NP = 10240
NC = 2
NS = 16
NW = NC * NS
K = 128
EPS = E // NS
CHT = 160
NQ = 4
CHQ = CHT // NQ
RPT = NP // NS
RCH = RPT // K
HR = NP // K


def _row_off(s, r):
    return pl.multiple_of(s * RPT + r * K, K)


NBUF = 2


def _conv_body(gb_half, vert_idx, edge_idx, zrow, out, iva, ivb, iea, ieb,
               rows_v, t_sh, a_sh, gsem0, gsem1, gsem2, gsem3, isem):
    c = lax.axis_index("c")
    s = lax.axis_index("s")
    gsem = (gsem0, gsem1)
    ssem = (gsem2, gsem3)
    col = pl.multiple_of(c * DH, 8)

    cps = []
    for r in range(RCH):
        off = _row_off(s, r)
        cps.append(pltpu.make_async_copy(
            gb_half.at[pl.ds(off, K), pl.ds(col, DH)],
            t_sh.at[pl.ds(off, K)], isem))
        cps.append(pltpu.make_async_copy(zrow, a_sh.at[pl.ds(off, K)], isem))
    for cp in cps:
        cp.start()
    for cp in cps:
        cp.wait()
    plsc.subcore_barrier()

    def phase(src_sh, src_idx, dst_sh, dst_idx):
        sbuf = (iva, ivb)
        dbuf = (iea, ieb)

        def i_cp(q, b):
            qo = pl.multiple_of(q * CHQ, CHQ)
            return (pltpu.make_async_copy(src_idx.at[s, pl.ds(qo, CHQ)],
                                          sbuf[b], isem),
                    pltpu.make_async_copy(dst_idx.at[s, pl.ds(qo, CHQ)],
                                          dbuf[b], isem))

        for cp in i_cp(0, 0):
            cp.start()
        for q in range(NQ):
            qb = q % 2
            for cp in i_cp(q, qb):
                cp.wait()
            if q + 1 < NQ:
                for cp in i_cp(q + 1, 1 - qb):
                    cp.start()
            src_v, dst_v = sbuf[qb], dbuf[qb]

            def g_cp(j, b):
                return pltpu.make_async_copy(src_sh.at[src_v.at[j]],
                                             rows_v.at[b], gsem[b])

            def s_cp(j, b):
                return pltpu.make_async_copy(rows_v.at[b],
                                             dst_sh.at[dst_v.at[j]], ssem[b])

            g_cp(0, 0).start()

            def pair(o, carry):
                for b in range(2):
                    j = o * 2 + b
                    g_cp(j, b).wait()
                    pltpu.async_copy(rows_v.at[b], dst_sh.at[dst_v.at[j]],
                                     ssem[b], add=True)

                    @pl.when(j + 1 < CHQ)
                    def _():
                        @pl.when(j >= 1)
                        def _():
                            s_cp(j - 1, 1 - b).wait()

                        g_cp(j + 1, 1 - b).start()
                return carry

            lax.fori_loop(0, CHQ // 2, pair, 0)
            s_cp(CHQ - 2, 0).wait()
            s_cp(CHQ - 1, 1).wait()

    phase(t_sh, vert_idx, a_sh, edge_idx)
    plsc.subcore_barrier()

    cps = [pltpu.make_async_copy(zrow, t_sh.at[pl.ds(_row_off(s, r), K)],
                                 isem) for r in range(RCH)]
    for cp in cps:
        cp.start()
    for cp in cps:
        cp.wait()
    plsc.subcore_barrier()

    phase(a_sh, edge_idx, t_sh, vert_idx)
    plsc.subcore_barrier()

    cps = []
    for r in range(RCH):
        off = _row_off(s, r)
        cps.append(pltpu.make_async_copy(
            t_sh.at[pl.ds(off, K)],
            out.at[pl.ds(off, K), pl.ds(col, DH)], isem))
    for cp in cps:
        cp.start()
    for cp in cps:
        cp.wait()


def _hist_body(vert_idx, out, idx_v, hacc):
    c = lax.axis_index("c")
    s = lax.axis_index("s")
    zeros16 = jnp.zeros((16,), jnp.float32)
    ones16 = jnp.ones((16,), jnp.float32)

    def zrow(r, carry):
        for g in range(K // 16):
            hacc[r, pl.ds(g * 16, 16)] = zeros16
        return carry

    lax.fori_loop(0, HR, zrow, 0)

    for q in range(NQ // NC):
        qq = q * NC
        pltpu.sync_copy(
            vert_idx.at[s, pl.ds(pl.multiple_of((qq + c) * CHQ, CHQ), CHQ)],
            idx_v)

        def chunk(j, carry):
            for g in range(K // 16):
                iv = idx_v[j, pl.ds(g * 16, 16)]
                row = lax.shift_right_logical(iv, 7)
                col = lax.bitwise_and(iv, 127)
                plsc.addupdate_scatter(hacc, [row, col], ones16)
            return carry

        lax.fori_loop(0, CHQ, chunk, 0)

    wid = c * NS + s
    pltpu.sync_copy(hacc, out.at[wid])


@functools.lru_cache(maxsize=None)
def _mesh():
    return plsc.VectorSubcoreMesh(core_axis_name="c", subcore_axis_name="s",
                                  num_cores=NC, num_subcores=NS)


@functools.lru_cache(maxsize=None)
def _make_conv():
    return pl.kernel(
        _conv_body,
        out_type=jax.ShapeDtypeStruct((NP, D), jnp.float32),
        mesh=_mesh(),
        scratch_types=[
            pltpu.VMEM((CHQ, K), jnp.int32),
            pltpu.VMEM((CHQ, K), jnp.int32),
            pltpu.VMEM((CHQ, K), jnp.int32),
            pltpu.VMEM((CHQ, K), jnp.int32),
            pltpu.VMEM((NBUF, K, DH), jnp.float32),
            pltpu.VMEM_SHARED((NP, DH), jnp.float32),
            pltpu.VMEM_SHARED((NP, DH), jnp.float32),
        ] + [pltpu.SemaphoreType.DMA] * 5,
        compiler_params=pltpu.CompilerParams(use_tc_tiling_on_sc=False),
    )


@functools.lru_cache(maxsize=None)
def _make_hist():
    return pl.kernel(
        _hist_body,
        out_type=jax.ShapeDtypeStruct((NW, HR, K), jnp.float32),
        mesh=_mesh(),
        scratch_types=[
            pltpu.VMEM((CHQ, K), jnp.int32),
            pltpu.VMEM((HR, K), jnp.float32),
        ],
        compiler_params=pltpu.CompilerParams(
            needs_layout_passes=False, use_tc_tiling_on_sc=False),
    )


def _t0_body(x_ref, wl_ref, bl_ref, w1_ref, b1_ref, w2a_ref, w2b_ref, b2_ref,
             h_ref, gb_ref, a_ref):
    h = jnp.maximum(
        jnp.dot(x_ref[...], wl_ref[...], preferred_element_type=jnp.float32)
        + bl_ref[...], 0.0)
    h_ref[...] = h
    g = jnp.dot(h, w1_ref[...], preferred_element_type=jnp.float32) + b1_ref[...]
    gb_ref[...] = jnp.dot(g, w2b_ref[...], preferred_element_type=jnp.float32)
    a_ref[...] = (jnp.dot(h, w2a_ref[...], preferred_element_type=jnp.float32)
                  + b2_ref[...])


def _hsum_body(hist_ref, cnt_ref):
    cnt_ref[...] = jnp.sum(hist_ref[...], axis=0)


def _mid_body(q_ref, cnt_ref, a_ref, h0_ref, w3_ref, b3_ref, w1_ref, b1_ref,
              w2a_ref, w2b_ref, b2_ref, gb_ref, a2_ref):
    xv = q_ref[...] + cnt_ref[...] * a_ref[...]
    u = 0.5 * xv + 0.5 * h0_ref[...]
    h2 = jnp.maximum(
        jnp.dot(u, w3_ref[...], preferred_element_type=jnp.float32)
        + b3_ref[...], 0.0)
    g = jnp.dot(h2, w1_ref[...], preferred_element_type=jnp.float32) + b1_ref[...]
    gb_ref[...] = jnp.dot(g, w2b_ref[...], preferred_element_type=jnp.float32)
    a2_ref[...] = (jnp.dot(h2, w2a_ref[...], preferred_element_type=jnp.float32)
                   + b2_ref[...])


def _final_body(q_ref, cnt_ref, a_ref, h0_ref, w3_ref, b3_ref, wc_ref, bc_ref,
                out_ref):
    xv = q_ref[...] + cnt_ref[...] * a_ref[...]
    u = 0.5 * xv + 0.5 * h0_ref[...]
    h3 = jnp.maximum(
        jnp.dot(u, w3_ref[...], preferred_element_type=jnp.float32)
        + b3_ref[...], 0.0)
    out_ref[...] = (jnp.dot(h3[:N], wc_ref[...],
                            preferred_element_type=jnp.float32) + bc_ref[...])


def _tc(body, out_shapes, *args):
    return pl.pallas_call(body, out_shape=out_shapes)(*args)


def kernel(x, edge_index, W_lin, b_lin, W1w, W1b, W2w, W2b, W3w, W3b, Wcw, Wcb):
    f32 = jnp.float32
    pad = jnp.full((NS, CHT * K - EPS), N, jnp.int32)
    vertex = jnp.concatenate(
        [edge_index[0].reshape(NS, EPS), pad], axis=1).reshape(NS, CHT, K)
    edges = jnp.concatenate(
        [edge_index[1].reshape(NS, EPS), pad], axis=1).reshape(NS, CHT, K)
    xp = jnp.pad(x, ((0, NP - N), (0, 0)))
    zrow = jnp.zeros((K, DH), f32)
    W2a, W2bb = W2w[:D], W2w[D:]
    bl = b_lin.reshape(1, D)
    b1 = W1b.reshape(1, D)
    b2 = W2b.reshape(1, D)
    b3 = W3b.reshape(1, D)
    bc = Wcb.reshape(1, -1)

    nd = jax.ShapeDtypeStruct((NP, D), f32)
    conv = _make_conv()

    h0, gb1, a1 = _tc(_t0_body, (nd, nd, nd),
                      xp, W_lin, bl, W1w, b1, W2a, W2bb, b2)
    hist = _make_hist()(vertex)
    cntm = _tc(_hsum_body, jax.ShapeDtypeStruct((HR, K), f32), hist)
    cnt_col = cntm.reshape(NP, 1)

    q1 = conv(gb1, vertex, edges, zrow)
    gb2, a2 = _tc(_mid_body, (nd, nd),
                  q1, cnt_col, a1, h0, W3w, b3, W1w, b1, W2a, W2bb, b2)

    q2 = conv(gb2, vertex, edges, zrow)
    out = _tc(_final_body, jax.ShapeDtypeStruct((N, Wcw.shape[1]), f32),
              q2, cnt_col, a2, h0, W3w, b3, Wcw, bc)
    return out
